# baseline (speedup 1.0000x reference)
import jax
import jax.numpy as jnp
from jax.experimental import pallas as pl
from jax.experimental.pallas import tpu as pltpu

_ROWS = 8

_K0 = 0
_K1 = 42
_KS2 = _K0 ^ _K1 ^ 0x1BD11BDA
_ROT = ((13, 15, 26, 6), (17, 29, 16, 24))
_TINY = 1.1754943508222875e-38


def _sample_kernel(x_ref, out_ref):
    pid = pl.program_id(0)
    x = x_ref[...]
    rows, c = x.shape

    col = jax.lax.broadcasted_iota(jnp.uint32, (rows, c), 1)
    row = jax.lax.broadcasted_iota(jnp.uint32, (rows, c), 0)
    flat = (row + pid.astype(jnp.uint32) * jnp.uint32(rows)) * jnp.uint32(c) + col

    ks = (jnp.uint32(_K0), jnp.uint32(_K1), jnp.uint32(_KS2))
    x0 = jnp.zeros_like(flat) + ks[0]
    x1 = flat + ks[1]
    for r in range(5):
        for rot in _ROT[r % 2]:
            x0 = x0 + x1
            x1 = (x1 << rot) | (x1 >> (32 - rot))
            x1 = x1 ^ x0
        x0 = x0 + ks[(r + 1) % 3]
        x1 = x1 + ks[(r + 2) % 3] + jnp.uint32(r + 1)
    bits = x0 ^ x1

    fb = (bits >> 9) | jnp.uint32(0x3F800000)
    f = jax.lax.bitcast_convert_type(fb, jnp.float32) - jnp.float32(1.0)
    u = jnp.maximum(jnp.float32(_TINY), f + jnp.float32(_TINY))
    gum = -jnp.log(-jnp.log(u))

    v = x + gum
    maxv = jnp.max(v, axis=1, keepdims=True)
    cand = jnp.where(v == maxv, col.astype(jnp.int32), jnp.int32(0x7FFFFFFF))
    tok = jnp.min(cand, axis=1)
    out_ref[0, 0, :] = tok


def kernel(logits):
    b, n, c = logits.shape
    x2 = logits.reshape(b * n, c)
    grid = (b * n) // _ROWS
    out = pl.pallas_call(
        _sample_kernel,
        grid=(grid,),
        in_specs=[pl.BlockSpec((_ROWS, c), lambda i: (i, 0))],
        out_specs=pl.BlockSpec((1, 1, _ROWS), lambda i: (i, 0, 0)),
        out_shape=jax.ShapeDtypeStruct((grid, 1, _ROWS), jnp.int32),
        compiler_params=pltpu.CompilerParams(
            dimension_semantics=("arbitrary",),
        ),
    )(x2)
    return out.reshape(b, n)


# speedup vs baseline: 1.0094x; 1.0094x over previous
import jax
import jax.numpy as jnp
from jax.experimental import pallas as pl
from jax.experimental.pallas import tpu as pltpu

_ROWS = 8

_K0 = 0
_K1 = 42
_KS2 = _K0 ^ _K1 ^ 0x1BD11BDA
_ROT = ((13, 15, 26, 6), (17, 29, 16, 24))
_TINY = 1.1754943508222875e-38


def _sample_kernel(x_ref, out_ref):
    pid = pl.program_id(0)
    x = x_ref[...]
    rows, c = x.shape

    col = jax.lax.broadcasted_iota(jnp.uint32, (rows, c), 1)
    row = jax.lax.broadcasted_iota(jnp.uint32, (rows, c), 0)
    flat = (row + pid.astype(jnp.uint32) * jnp.uint32(rows)) * jnp.uint32(c) + col

    ks = (jnp.uint32(_K0), jnp.uint32(_K1), jnp.uint32(_KS2))
    x0 = jnp.zeros_like(flat) + ks[0]
    x1 = flat + ks[1]
    for r in range(5):
        for rot in _ROT[r % 2]:
            x0 = x0 + x1
            x1 = (x1 << rot) | (x1 >> (32 - rot))
            x1 = x1 ^ x0
        x0 = x0 + ks[(r + 1) % 3]
        x1 = x1 + ks[(r + 2) % 3] + jnp.uint32(r + 1)
    bits = x0 ^ x1

    fb = (bits >> 9) | jnp.uint32(0x3F800000)
    f = jax.lax.bitcast_convert_type(fb, jnp.float32) - jnp.float32(1.0)
    u = jnp.maximum(jnp.float32(_TINY), f + jnp.float32(_TINY))
    gum = -jnp.log(-jnp.log(u))

    v = x + gum
    maxv = jnp.max(v, axis=1, keepdims=True)
    cand = jnp.where(v == maxv, col.astype(jnp.int32), jnp.int32(0x7FFFFFFF))
    tok = jnp.min(cand, axis=1)
    out_ref[0, 0, :] = tok


def kernel(logits):
    b, n, c = logits.shape
    x2 = logits.reshape(b * n, c)
    grid = (b * n) // _ROWS
    out = pl.pallas_call(
        _sample_kernel,
        grid=(grid,),
        in_specs=[pl.BlockSpec((_ROWS, c), lambda i: (i, 0))],
        out_specs=pl.BlockSpec((1, 1, _ROWS), lambda i: (i, 0, 0)),
        out_shape=jax.ShapeDtypeStruct((grid, 1, _ROWS), jnp.int32),
        compiler_params=pltpu.CompilerParams(
            dimension_semantics=("parallel",),
        ),
    )(x2)
    return out.reshape(b, n)
